# baseline (speedup 1.0000x reference)
import functools

import jax
import jax.numpy as jnp
from jax import lax
from jax.experimental import pallas as pl
from jax.experimental.pallas import tpu as pltpu
from jax.experimental.pallas import tpu_sc as plsc

NUM_EXPERTS = 8
TOP_K = 2
BS = 256
SC_CORES = 2
SC_SUBCORES = 16
NW = SC_CORES * SC_SUBCORES


def _router_kernel(x_ref, gw_ref, ei_ref, ew_ref):
    x = x_ref[...]
    gw = gw_ref[...]
    logits = jax.lax.dot_general(
        x, gw, (((1,), (1,)), ((), ())),
        preferred_element_type=jnp.float32,
        precision=jax.lax.Precision.DEFAULT)
    p = jax.nn.softmax(logits, axis=-1)
    e_dim = p.shape[-1]
    iota = jax.lax.broadcasted_iota(jnp.int32, p.shape, 1)
    m1 = jnp.max(p, axis=-1, keepdims=True)
    i1 = jnp.min(jnp.where(p == m1, iota, e_dim), axis=-1, keepdims=True)
    mask1 = iota == i1
    pm = jnp.where(mask1, -jnp.inf, p)
    m2 = jnp.max(pm, axis=-1, keepdims=True)
    i2 = jnp.min(jnp.where(pm == m2, iota, e_dim), axis=-1, keepdims=True)
    denom = m1 + m2
    ei_ref[...] = jnp.concatenate([i1, i2], axis=1)
    ew_ref[...] = jnp.concatenate([m1 / denom, m2 / denom], axis=1)


def _gmm_kernel(be_ref, nv_ref, first_ref, slot_ref, pfe_ref,
                xs_ref, gp_ref, up_ref, dp_ref, ys_ref,
                gpc_ref, upc_ref, dpc_ref, sems):
    i = pl.program_id(0)

    def _start(e, par):
        pltpu.make_async_copy(
            gp_ref.at[e], gpc_ref.at[par], sems.at[par]).start()
        pltpu.make_async_copy(
            up_ref.at[e], upc_ref.at[par], sems.at[par]).start()
        pltpu.make_async_copy(
            dp_ref.at[e], dpc_ref.at[par], sems.at[par]).start()

    def _wait(e, par):
        pltpu.make_async_copy(
            gp_ref.at[e], gpc_ref.at[par], sems.at[par]).wait()
        pltpu.make_async_copy(
            up_ref.at[e], upc_ref.at[par], sems.at[par]).wait()
        pltpu.make_async_copy(
            dp_ref.at[e], dpc_ref.at[par], sems.at[par]).wait()

    @pl.when(i < nv_ref[0])
    def _():
        e = be_ref[i]
        par = slot_ref[i]

        @pl.when(i == 0)
        def _():
            _start(e, par)

        @pl.when(first_ref[i] == 1)
        def _():
            _wait(e, par)
            pe = pfe_ref[i]

            @pl.when(pe >= 0)
            def _():
                _start(pe, 1 - par)

        xb = xs_ref[...]
        g = jax.lax.dot_general(
            xb, gpc_ref[par], (((1,), (1,)), ((), ())),
            preferred_element_type=jnp.float32,
            precision=jax.lax.Precision.DEFAULT)
        u = jax.lax.dot_general(
            xb, upc_ref[par], (((1,), (1,)), ((), ())),
            preferred_element_type=jnp.float32,
            precision=jax.lax.Precision.DEFAULT)
        h = (g * jax.lax.logistic(g)) * u
        y = jax.lax.dot_general(
            h, dpc_ref[par], (((1,), (1,)), ((), ())),
            preferred_element_type=jnp.float32,
            precision=jax.lax.Precision.DEFAULT)
        ys_ref[...] = y


def _wadd_kernel(yg0_ref, yg1_ref, w_ref, out_ref):
    w = w_ref[...]
    out_ref[...] = w[:, 0:1] * yg0_ref[...] + w[:, 1:2] * yg1_ref[...]


def _sc_dispatch(x, tok3, dst3, p_max, d, n_pairs):
    _, k_chunks, ck = tok3.shape
    mesh = plsc.VectorSubcoreMesh(core_axis_name="c", subcore_axis_name="s")

    @functools.partial(
        pl.kernel, mesh=mesh,
        out_type=jax.ShapeDtypeStruct((p_max, d), jnp.float32),
        scratch_types=[
            pltpu.VMEM((k_chunks, ck), jnp.int32),
            pltpu.VMEM((k_chunks, ck), jnp.int32),
            pltpu.VMEM((ck, d), jnp.float32),
            pltpu.VMEM((ck, d), jnp.float32),
            pltpu.SemaphoreType.DMA,
            pltpu.SemaphoreType.DMA,
        ],
    )
    def k(x_hbm, tok_hbm, dst_hbm, xs_hbm, tok_v, dst_v, rows_a, rows_b,
          sem_a, sem_b):
        wid = lax.axis_index("s") * SC_CORES + lax.axis_index("c")
        pltpu.sync_copy(tok_hbm.at[wid], tok_v)
        pltpu.sync_copy(dst_hbm.at[wid], dst_v)

        bufs = (rows_a, rows_b)
        sems = (sem_a, sem_b)
        pend = {}
        for ci in range(k_chunks):
            if ci >= 2:
                pend.pop(ci - 2).wait()
                pltpu.sync_copy(bufs[(ci - 2) % 2],
                                xs_hbm.at[dst_v.at[ci - 2]])
            pend[ci] = pltpu.async_copy(
                x_hbm.at[tok_v.at[ci]], bufs[ci % 2], sems[ci % 2])
        for ci in sorted(pend):
            pend[ci].wait()
            pltpu.sync_copy(bufs[ci % 2], xs_hbm.at[dst_v.at[ci]])

    return k(x, tok3, dst3)


def _sc_gather(ys, idx, n_rows, d):
    b_per_w = n_rows // NW
    cr = 32
    n_chunks = b_per_w // cr
    mesh = plsc.VectorSubcoreMesh(core_axis_name="c", subcore_axis_name="s")

    @functools.partial(
        pl.kernel, mesh=mesh,
        out_type=jax.ShapeDtypeStruct((n_rows, d), jnp.float32),
        scratch_types=[
            pltpu.VMEM((b_per_w,), jnp.int32),
            pltpu.VMEM((cr, d), jnp.float32),
            pltpu.VMEM((cr, d), jnp.float32),
            pltpu.SemaphoreType.DMA,
            pltpu.SemaphoreType.DMA,
        ],
    )
    def k(ys_hbm, idx_hbm, out_hbm, idx_v, rows_a, rows_b, sem_a, sem_b):
        wid = lax.axis_index("s") * SC_CORES + lax.axis_index("c")
        base = wid * b_per_w
        pltpu.sync_copy(idx_hbm.at[pl.ds(base, b_per_w)], idx_v)

        bufs = (rows_a, rows_b)
        sems = (sem_a, sem_b)
        pend = {}
        for ci in range(n_chunks):
            if ci >= 2:
                pend.pop(ci - 2).wait()
                pltpu.sync_copy(
                    bufs[ci % 2],
                    out_hbm.at[pl.ds(base + (ci - 2) * cr, cr)])
            pend[ci] = pltpu.async_copy(
                ys_hbm.at[idx_v.at[pl.ds(ci * cr, cr)]],
                bufs[ci % 2], sems[ci % 2])
        for ci in sorted(pend):
            pend[ci].wait()
            pltpu.sync_copy(bufs[ci % 2],
                            out_hbm.at[pl.ds(base + ci * cr, cr)])

    return k(ys, idx)


@functools.partial(jax.jit, static_argnames=())
def kernel(hidden_states, gate_w, gate_proj, up_proj, down_proj):
    b, s, h = hidden_states.shape
    x = hidden_states.reshape(-1, h)
    t = x.shape[0]
    f = gate_proj.shape[1]
    n_pairs = t * TOP_K
    p_max = n_pairs + NUM_EXPERTS * BS
    nb = p_max // BS

    ei, ew = pl.pallas_call(
        _router_kernel,
        out_shape=(jax.ShapeDtypeStruct((t, TOP_K), jnp.int32),
                   jax.ShapeDtypeStruct((t, TOP_K), jnp.float32)),
    )(x, gate_w)

    eflat = ei.reshape(-1)
    onehot = (eflat[:, None] == jnp.arange(NUM_EXPERTS)[None, :]
              ).astype(jnp.int32)
    ranks = jnp.cumsum(onehot, axis=0) - 1
    myrank = jnp.take_along_axis(ranks, eflat[:, None], axis=1)[:, 0]
    counts = onehot.sum(axis=0)
    padded = ((counts + BS - 1) // BS) * BS
    cum_pad = jnp.cumsum(padded)
    pad_off = cum_pad - padded
    dst = pad_off[eflat] + myrank
    tok = jnp.arange(n_pairs, dtype=jnp.int32) // TOP_K
    blk_start = jnp.arange(nb, dtype=jnp.int32) * BS

    ck = 32
    k_chunks = n_pairs // (NW * ck)
    tok3 = tok.reshape(NW, k_chunks, ck)
    dst3 = dst.reshape(NW, k_chunks, ck)

    xs = _sc_dispatch(x, tok3, dst3, p_max, h, n_pairs)

    be = jnp.minimum(
        jnp.sum(blk_start[:, None] >= cum_pad[None, :], axis=1),
        NUM_EXPERTS - 1).astype(jnp.int32)
    nvalid = (cum_pad[-1] // BS).astype(jnp.int32)[None]
    first = jnp.concatenate(
        [jnp.ones((1,), jnp.int32),
         (be[1:] != be[:-1]).astype(jnp.int32)])
    slot = ((jnp.cumsum(first) - 1) % 2).astype(jnp.int32)
    run_end = jnp.sum(
        be[:, None] >= be[None, :], axis=1).astype(jnp.int32)
    pfe = jnp.where(run_end < nvalid[0],
                    be[jnp.minimum(run_end, nb - 1)], -1).astype(jnp.int32)

    grid_spec = pltpu.PrefetchScalarGridSpec(
        num_scalar_prefetch=5,
        grid=(nb,),
        in_specs=[
            pl.BlockSpec((BS, h), lambda i, *_: (i, 0)),
            pl.BlockSpec(memory_space=pl.ANY),
            pl.BlockSpec(memory_space=pl.ANY),
            pl.BlockSpec(memory_space=pl.ANY),
        ],
        out_specs=pl.BlockSpec((BS, h), lambda i, *_: (i, 0)),
        scratch_shapes=[
            pltpu.VMEM((2, f, h), jnp.float32),
            pltpu.VMEM((2, f, h), jnp.float32),
            pltpu.VMEM((2, h, f), jnp.float32),
            pltpu.SemaphoreType.DMA((2,)),
        ],
    )
    ys = pl.pallas_call(
        _gmm_kernel,
        grid_spec=grid_spec,
        out_shape=jax.ShapeDtypeStruct((p_max, h), jnp.float32),
    )(be, nvalid, first, slot, pfe, xs, gate_proj, up_proj, down_proj)

    dst_kt = dst.reshape(t, TOP_K).T.reshape(-1)
    yg = _sc_gather(ys, dst_kt, n_pairs, h)

    tb = 256
    out = pl.pallas_call(
        _wadd_kernel,
        grid=(t // tb,),
        in_specs=[pl.BlockSpec((tb, h), lambda i: (i, 0)),
                  pl.BlockSpec((tb, h), lambda i: (i + t // tb, 0)),
                  pl.BlockSpec((tb, TOP_K), lambda i: (i, 0))],
        out_specs=pl.BlockSpec((tb, h), lambda i: (i, 0)),
        out_shape=jax.ShapeDtypeStruct((t, h), jnp.float32),
    )(yg, yg, ew)
    return out.reshape(b, s, h)


# speedup vs baseline: 1.0632x; 1.0632x over previous
import functools

import jax
import jax.numpy as jnp
from jax import lax
from jax.experimental import pallas as pl
from jax.experimental.pallas import tpu as pltpu
from jax.experimental.pallas import tpu_sc as plsc

NUM_EXPERTS = 8
TOP_K = 2
BS = 256
SC_CORES = 2
SC_SUBCORES = 16
NW = SC_CORES * SC_SUBCORES


def _router_kernel(x_ref, gw_ref, ei_ref, ew_ref):
    x = x_ref[...]
    gw = gw_ref[...]
    logits = jax.lax.dot_general(
        x, gw, (((1,), (1,)), ((), ())),
        preferred_element_type=jnp.float32,
        precision=jax.lax.Precision.DEFAULT)
    p = jax.nn.softmax(logits, axis=-1)
    e_dim = p.shape[-1]
    iota = jax.lax.broadcasted_iota(jnp.int32, p.shape, 1)
    m1 = jnp.max(p, axis=-1, keepdims=True)
    i1 = jnp.min(jnp.where(p == m1, iota, e_dim), axis=-1, keepdims=True)
    mask1 = iota == i1
    pm = jnp.where(mask1, -jnp.inf, p)
    m2 = jnp.max(pm, axis=-1, keepdims=True)
    i2 = jnp.min(jnp.where(pm == m2, iota, e_dim), axis=-1, keepdims=True)
    denom = m1 + m2
    ei_ref[...] = jnp.concatenate([i1, i2], axis=1)
    ew_ref[...] = jnp.concatenate([m1 / denom, m2 / denom], axis=1)


def _gmm_kernel(be_ref, nv_ref, first_ref, slot_ref, pfe_ref,
                xs_ref, gp_ref, up_ref, dp_ref, ys_ref,
                gpc_ref, upc_ref, dpc_ref, sems):
    i = pl.program_id(0)

    def _start(e, par):
        pltpu.make_async_copy(
            gp_ref.at[e], gpc_ref.at[par], sems.at[par]).start()
        pltpu.make_async_copy(
            up_ref.at[e], upc_ref.at[par], sems.at[par]).start()
        pltpu.make_async_copy(
            dp_ref.at[e], dpc_ref.at[par], sems.at[par]).start()

    def _wait(e, par):
        pltpu.make_async_copy(
            gp_ref.at[e], gpc_ref.at[par], sems.at[par]).wait()
        pltpu.make_async_copy(
            up_ref.at[e], upc_ref.at[par], sems.at[par]).wait()
        pltpu.make_async_copy(
            dp_ref.at[e], dpc_ref.at[par], sems.at[par]).wait()

    @pl.when(i < nv_ref[0])
    def _():
        e = be_ref[i]
        par = slot_ref[i]

        @pl.when(i == 0)
        def _():
            _start(e, par)

        @pl.when(first_ref[i] == 1)
        def _():
            _wait(e, par)
            pe = pfe_ref[i]

            @pl.when(pe >= 0)
            def _():
                _start(pe, 1 - par)

        xb = xs_ref[...]
        g = jax.lax.dot_general(
            xb, gpc_ref[par], (((1,), (1,)), ((), ())),
            preferred_element_type=jnp.float32,
            precision=jax.lax.Precision.DEFAULT)
        u = jax.lax.dot_general(
            xb, upc_ref[par], (((1,), (1,)), ((), ())),
            preferred_element_type=jnp.float32,
            precision=jax.lax.Precision.DEFAULT)
        h = (g * jax.lax.logistic(g)) * u
        y = jax.lax.dot_general(
            h, dpc_ref[par], (((1,), (1,)), ((), ())),
            preferred_element_type=jnp.float32,
            precision=jax.lax.Precision.DEFAULT)
        ys_ref[...] = y


def _wadd_kernel(yg0_ref, yg1_ref, w_ref, out_ref):
    w = w_ref[...]
    out_ref[...] = w[:, 0:1] * yg0_ref[...] + w[:, 1:2] * yg1_ref[...]


def _sc_dispatch(x, dst_e3, dst_o3, p_max, d):
    _, k_chunks, ck = dst_e3.shape
    mesh = plsc.VectorSubcoreMesh(core_axis_name="c", subcore_axis_name="s")

    @functools.partial(
        pl.kernel, mesh=mesh,
        out_type=jax.ShapeDtypeStruct((p_max, d), jnp.float32),
        scratch_types=[
            pltpu.VMEM((k_chunks, ck), jnp.int32),
            pltpu.VMEM((k_chunks, ck), jnp.int32),
            pltpu.VMEM((ck, d), jnp.float32),
            pltpu.VMEM((ck, d), jnp.float32),
            pltpu.SemaphoreType.DMA,
            pltpu.SemaphoreType.DMA,
        ],
    )
    def k(x_hbm, de_hbm, do_hbm, xs_hbm, de_v, do_v, rows_a, rows_b,
          sem_a, sem_b):
        wid = lax.axis_index("s") * SC_CORES + lax.axis_index("c")
        base = wid * k_chunks * ck
        pltpu.sync_copy(de_hbm.at[wid], de_v)
        pltpu.sync_copy(do_hbm.at[wid], do_v)

        bufs = (rows_a, rows_b)
        sems = (sem_a, sem_b)
        pend = {}
        for ci in range(k_chunks):
            if ci >= 2:
                pend.pop(ci - 2).wait()
                pltpu.sync_copy(bufs[(ci - 2) % 2],
                                xs_hbm.at[de_v.at[ci - 2]])
                pltpu.sync_copy(bufs[(ci - 2) % 2],
                                xs_hbm.at[do_v.at[ci - 2]])
            pend[ci] = pltpu.async_copy(
                x_hbm.at[pl.ds(base + ci * ck, ck)],
                bufs[ci % 2], sems[ci % 2])
        for ci in sorted(pend):
            pend[ci].wait()
            pltpu.sync_copy(bufs[ci % 2], xs_hbm.at[de_v.at[ci]])
            pltpu.sync_copy(bufs[ci % 2], xs_hbm.at[do_v.at[ci]])

    return k(x, dst_e3, dst_o3)


def _sc_gather(ys, idx, n_rows, d):
    b_per_w = n_rows // NW
    cr = 32
    n_chunks = b_per_w // cr
    mesh = plsc.VectorSubcoreMesh(core_axis_name="c", subcore_axis_name="s")

    @functools.partial(
        pl.kernel, mesh=mesh,
        out_type=jax.ShapeDtypeStruct((n_rows, d), jnp.float32),
        scratch_types=[
            pltpu.VMEM((b_per_w,), jnp.int32),
            pltpu.VMEM((cr, d), jnp.float32),
            pltpu.VMEM((cr, d), jnp.float32),
            pltpu.SemaphoreType.DMA,
            pltpu.SemaphoreType.DMA,
        ],
    )
    def k(ys_hbm, idx_hbm, out_hbm, idx_v, rows_a, rows_b, sem_a, sem_b):
        wid = lax.axis_index("s") * SC_CORES + lax.axis_index("c")
        base = wid * b_per_w
        pltpu.sync_copy(idx_hbm.at[pl.ds(base, b_per_w)], idx_v)

        bufs = (rows_a, rows_b)
        sems = (sem_a, sem_b)
        pend = {}
        for ci in range(n_chunks):
            if ci >= 2:
                pend.pop(ci - 2).wait()
                pltpu.sync_copy(
                    bufs[ci % 2],
                    out_hbm.at[pl.ds(base + (ci - 2) * cr, cr)])
            pend[ci] = pltpu.async_copy(
                ys_hbm.at[idx_v.at[pl.ds(ci * cr, cr)]],
                bufs[ci % 2], sems[ci % 2])
        for ci in sorted(pend):
            pend[ci].wait()
            pltpu.sync_copy(bufs[ci % 2],
                            out_hbm.at[pl.ds(base + ci * cr, cr)])

    return k(ys, idx)


@functools.partial(jax.jit, static_argnames=())
def kernel(hidden_states, gate_w, gate_proj, up_proj, down_proj):
    b, s, h = hidden_states.shape
    x = hidden_states.reshape(-1, h)
    t = x.shape[0]
    f = gate_proj.shape[1]
    n_pairs = t * TOP_K
    p_max = n_pairs + NUM_EXPERTS * BS
    nb = p_max // BS

    ei, ew = pl.pallas_call(
        _router_kernel,
        out_shape=(jax.ShapeDtypeStruct((t, TOP_K), jnp.int32),
                   jax.ShapeDtypeStruct((t, TOP_K), jnp.float32)),
    )(x, gate_w)

    eflat = ei.reshape(-1)
    onehot = (eflat[:, None] == jnp.arange(NUM_EXPERTS)[None, :]
              ).astype(jnp.int32)
    ranks = jnp.cumsum(onehot, axis=0) - 1
    myrank = jnp.take_along_axis(ranks, eflat[:, None], axis=1)[:, 0]
    counts = onehot.sum(axis=0)
    padded = ((counts + BS - 1) // BS) * BS
    cum_pad = jnp.cumsum(padded)
    pad_off = cum_pad - padded
    dst = pad_off[eflat] + myrank
    blk_start = jnp.arange(nb, dtype=jnp.int32) * BS

    ck = 32
    k_chunks = t // (NW * ck)
    dst_e3 = dst[0::2].reshape(NW, k_chunks, ck)
    dst_o3 = dst[1::2].reshape(NW, k_chunks, ck)

    xs = _sc_dispatch(x, dst_e3, dst_o3, p_max, h)

    be = jnp.minimum(
        jnp.sum(blk_start[:, None] >= cum_pad[None, :], axis=1),
        NUM_EXPERTS - 1).astype(jnp.int32)
    nvalid = (cum_pad[-1] // BS).astype(jnp.int32)[None]
    first = jnp.concatenate(
        [jnp.ones((1,), jnp.int32),
         (be[1:] != be[:-1]).astype(jnp.int32)])
    slot = ((jnp.cumsum(first) - 1) % 2).astype(jnp.int32)
    run_end = jnp.sum(
        be[:, None] >= be[None, :], axis=1).astype(jnp.int32)
    pfe = jnp.where(run_end < nvalid[0],
                    be[jnp.minimum(run_end, nb - 1)], -1).astype(jnp.int32)

    grid_spec = pltpu.PrefetchScalarGridSpec(
        num_scalar_prefetch=5,
        grid=(nb,),
        in_specs=[
            pl.BlockSpec((BS, h), lambda i, *_: (i, 0)),
            pl.BlockSpec(memory_space=pl.ANY),
            pl.BlockSpec(memory_space=pl.ANY),
            pl.BlockSpec(memory_space=pl.ANY),
        ],
        out_specs=pl.BlockSpec((BS, h), lambda i, *_: (i, 0)),
        scratch_shapes=[
            pltpu.VMEM((2, f, h), jnp.float32),
            pltpu.VMEM((2, f, h), jnp.float32),
            pltpu.VMEM((2, h, f), jnp.float32),
            pltpu.SemaphoreType.DMA((2,)),
        ],
    )
    ys = pl.pallas_call(
        _gmm_kernel,
        grid_spec=grid_spec,
        out_shape=jax.ShapeDtypeStruct((p_max, h), jnp.float32),
    )(be, nvalid, first, slot, pfe, xs, gate_proj, up_proj, down_proj)

    dst_kt = dst.reshape(t, TOP_K).T.reshape(-1)
    yg = _sc_gather(ys, dst_kt, n_pairs, h)

    tb = 256
    out = pl.pallas_call(
        _wadd_kernel,
        grid=(t // tb,),
        in_specs=[pl.BlockSpec((tb, h), lambda i: (i, 0)),
                  pl.BlockSpec((tb, h), lambda i: (i + t // tb, 0)),
                  pl.BlockSpec((tb, TOP_K), lambda i: (i, 0))],
        out_specs=pl.BlockSpec((tb, h), lambda i: (i, 0)),
        out_shape=jax.ShapeDtypeStruct((t, h), jnp.float32),
    )(yg, yg, ew)
    return out.reshape(b, s, h)
